# initial kernel scaffold (unmeasured)
import jax
import jax.numpy as jnp
from jax import lax
from jax.experimental import pallas as pl
from jax.experimental.pallas import tpu as pltpu

B, SQ, H, D = 4, 32, 8, 128
SKV_LOCAL = 4096
N_REPL = 4
SKV_Q = SKV_LOCAL // N_REPL
SCALE = D ** -0.5
N_STEPS = 3


def kernel(Q, K, V):
    def body(q_ref, k_hbm, v_hbm, out_ref,
             kq, vq, o_acc, ml_acc, o_send, o_recv, ml_send, ml_recv,
             load_sems, send_sems, recv_sems):
        mx = lax.axis_index("x")
        my = lax.axis_index("y")
        mz = lax.axis_index("z")
        r = mx * 2 + mz

        start = r * SKV_Q
        k_copy = pltpu.make_async_copy(
            k_hbm.at[:, pl.ds(start, SKV_Q)], kq, load_sems.at[0])
        v_copy = pltpu.make_async_copy(
            v_hbm.at[:, pl.ds(start, SKV_Q)], vq, load_sems.at[1])
        k_copy.start()
        v_copy.start()

        partners = [
            (mx, my, 1 - mz),
            (mx, 1 - my, mz),
            (1 - mx, my, mz),
        ]
        barrier = pltpu.get_barrier_semaphore()
        for p in partners:
            pl.semaphore_signal(barrier, inc=1, device_id=p,
                                device_id_type=pl.DeviceIdType.MESH)
        pl.semaphore_wait(barrier, N_STEPS)

        k_copy.wait()
        v_copy.wait()

        for b in range(B):
            for h in range(H):
                q = q_ref[b, :, h, :].astype(jnp.bfloat16)
                k = kq[b, :, h, :].astype(jnp.bfloat16)
                v = vq[b, :, h, :].astype(jnp.bfloat16)
                s = lax.dot_general(
                    q, k, (((1,), (1,)), ((), ())),
                    preferred_element_type=jnp.float32) * SCALE
                m = jnp.max(s, axis=1, keepdims=True)
                p = jnp.exp(s - m)
                l = jnp.sum(p, axis=1, keepdims=True)
                o = lax.dot_general(
                    p.astype(jnp.bfloat16), v, (((1,), (0,)), ((), ())),
                    preferred_element_type=jnp.float32)
                o_acc[b, h] = o
                ml_acc[b, h, :, 0:1] = m
                ml_acc[b, h, :, 1:2] = l

        for step in range(N_STEPS):
            o_send[step] = o_acc[...].astype(jnp.bfloat16)
            ml_send[step] = ml_acc[...]
            rdma_o = pltpu.make_async_remote_copy(
                src_ref=o_send.at[step], dst_ref=o_recv.at[step],
                send_sem=send_sems.at[step, 0], recv_sem=recv_sems.at[step, 0],
                device_id=partners[step],
                device_id_type=pl.DeviceIdType.MESH)
            rdma_ml = pltpu.make_async_remote_copy(
                src_ref=ml_send.at[step], dst_ref=ml_recv.at[step],
                send_sem=send_sems.at[step, 1], recv_sem=recv_sems.at[step, 1],
                device_id=partners[step],
                device_id_type=pl.DeviceIdType.MESH)
            rdma_o.start()
            rdma_ml.start()
            rdma_o.wait()
            rdma_ml.wait()

            m_a = ml_acc[:, :, :, 0:1]
            l_a = ml_acc[:, :, :, 1:2]
            m_b = ml_recv[step, :, :, :, 0:1]
            l_b = ml_recv[step, :, :, :, 1:2]
            m_n = jnp.maximum(m_a, m_b)
            ea = jnp.exp(m_a - m_n)
            eb = jnp.exp(m_b - m_n)
            o_acc[...] = ea * o_acc[...] + eb * o_recv[step].astype(jnp.float32)
            ml_acc[:, :, :, 0:1] = m_n
            ml_acc[:, :, :, 1:2] = ea * l_a + eb * l_b

        for b in range(B):
            for h in range(H):
                out_ref[b, :, h, :] = o_acc[b, h] / ml_acc[b, h, :, 1:2]

    return pl.pallas_call(
        body,
        out_shape=jax.ShapeDtypeStruct((B, SQ, H, D), jnp.float32),
        in_specs=[
            pl.BlockSpec(memory_space=pltpu.VMEM),
            pl.BlockSpec(memory_space=pltpu.ANY),
            pl.BlockSpec(memory_space=pltpu.ANY),
        ],
        out_specs=pl.BlockSpec(memory_space=pltpu.VMEM),
        scratch_shapes=[
            pltpu.VMEM((B, SKV_Q, H, D), jnp.float32),
            pltpu.VMEM((B, SKV_Q, H, D), jnp.float32),
            pltpu.VMEM((B, H, SQ, D), jnp.float32),
            pltpu.VMEM((B, H, SQ, 2), jnp.float32),
            pltpu.VMEM((N_STEPS, B, H, SQ, D), jnp.bfloat16),
            pltpu.VMEM((N_STEPS, B, H, SQ, D), jnp.bfloat16),
            pltpu.VMEM((N_STEPS, B, H, SQ, 2), jnp.float32),
            pltpu.VMEM((N_STEPS, B, H, SQ, 2), jnp.float32),
            pltpu.SemaphoreType.DMA((2,)),
            pltpu.SemaphoreType.DMA((N_STEPS, 2)),
            pltpu.SemaphoreType.DMA((N_STEPS, 2)),
        ],
        compiler_params=pltpu.CompilerParams(collective_id=0),
    )(Q, K, V)


# baseline (device time: 88097 ns/iter reference)
import jax
import jax.numpy as jnp
from jax import lax
from jax.experimental import pallas as pl
from jax.experimental.pallas import tpu as pltpu

B, SQ, H, D = 4, 32, 8, 128
SKV_LOCAL = 4096
N_REPL = 4
SKV_Q = SKV_LOCAL // N_REPL
SCALE = D ** -0.5
N_STEPS = 3


def kernel(Q, K, V):
    def body(q_ref, k_hbm, v_hbm, out_ref,
             kq, vq, o_acc, ml_acc, o_send, o_recv, ml_send, ml_recv,
             load_sems, send_sems, recv_sems):
        mx = lax.axis_index("x")
        my = lax.axis_index("y")
        mz = lax.axis_index("z")
        r = mx * 2 + mz

        start = r * SKV_Q
        k_copy = pltpu.make_async_copy(
            k_hbm.at[:, pl.ds(start, SKV_Q)], kq, load_sems.at[0])
        v_copy = pltpu.make_async_copy(
            v_hbm.at[:, pl.ds(start, SKV_Q)], vq, load_sems.at[1])
        k_copy.start()
        v_copy.start()

        partners = [
            (mx, my, 1 - mz),
            (mx, 1 - my, mz),
            (1 - mx, my, mz),
        ]
        barrier = pltpu.get_barrier_semaphore()
        for p in partners:
            pl.semaphore_signal(barrier, inc=1, device_id=p,
                                device_id_type=pl.DeviceIdType.MESH)
        pl.semaphore_wait(barrier, N_STEPS)

        k_copy.wait()
        v_copy.wait()

        for b in range(B):
            for h in range(H):
                q = q_ref[b, :, h, :].astype(jnp.bfloat16)
                k = kq[b, :, h, :].astype(jnp.bfloat16)
                v = vq[b, :, h, :].astype(jnp.bfloat16)
                s = lax.dot_general(
                    q, k, (((1,), (1,)), ((), ())),
                    preferred_element_type=jnp.float32) * SCALE
                m = jnp.max(s, axis=1, keepdims=True)
                p = jnp.exp(s - m)
                l = jnp.sum(p, axis=1, keepdims=True)
                o = lax.dot_general(
                    p.astype(jnp.bfloat16), v, (((1,), (0,)), ((), ())),
                    preferred_element_type=jnp.float32)
                o_acc[b, h] = o
                ml_acc[b, h, :, 0:1] = m
                ml_acc[b, h, :, 1:2] = l

        for step in range(N_STEPS):
            o_send[step] = o_acc[...].astype(jnp.bfloat16)
            ml_send[step] = ml_acc[...]
            rdma_o = pltpu.make_async_remote_copy(
                src_ref=o_send.at[step], dst_ref=o_recv.at[step],
                send_sem=send_sems.at[step, 0], recv_sem=recv_sems.at[step, 0],
                device_id=partners[step],
                device_id_type=pl.DeviceIdType.MESH)
            rdma_ml = pltpu.make_async_remote_copy(
                src_ref=ml_send.at[step], dst_ref=ml_recv.at[step],
                send_sem=send_sems.at[step, 1], recv_sem=recv_sems.at[step, 1],
                device_id=partners[step],
                device_id_type=pl.DeviceIdType.MESH)
            rdma_o.start()
            rdma_ml.start()
            rdma_o.wait()
            rdma_ml.wait()

            m_a = ml_acc[:, :, :, 0:1]
            l_a = ml_acc[:, :, :, 1:2]
            m_b = ml_recv[step, :, :, :, 0:1]
            l_b = ml_recv[step, :, :, :, 1:2]
            m_n = jnp.maximum(m_a, m_b)
            ea = jnp.exp(m_a - m_n)
            eb = jnp.exp(m_b - m_n)
            o_acc[...] = ea * o_acc[...] + eb * o_recv[step].astype(jnp.float32)
            ml_acc[:, :, :, 0:1] = m_n
            ml_acc[:, :, :, 1:2] = ea * l_a + eb * l_b

        for b in range(B):
            for h in range(H):
                out_ref[b, :, h, :] = o_acc[b, h] / ml_acc[b, h, :, 1:2]

    return pl.pallas_call(
        body,
        out_shape=jax.ShapeDtypeStruct((B, SQ, H, D), jnp.float32),
        in_specs=[
            pl.BlockSpec(memory_space=pltpu.VMEM),
            pl.BlockSpec(memory_space=pl.ANY),
            pl.BlockSpec(memory_space=pl.ANY),
        ],
        out_specs=pl.BlockSpec(memory_space=pltpu.VMEM),
        scratch_shapes=[
            pltpu.VMEM((B, SKV_Q, H, D), jnp.float32),
            pltpu.VMEM((B, SKV_Q, H, D), jnp.float32),
            pltpu.VMEM((B, H, SQ, D), jnp.float32),
            pltpu.VMEM((B, H, SQ, 2), jnp.float32),
            pltpu.VMEM((N_STEPS, B, H, SQ, D), jnp.bfloat16),
            pltpu.VMEM((N_STEPS, B, H, SQ, D), jnp.bfloat16),
            pltpu.VMEM((N_STEPS, B, H, SQ, 2), jnp.float32),
            pltpu.VMEM((N_STEPS, B, H, SQ, 2), jnp.float32),
            pltpu.SemaphoreType.DMA((2,)),
            pltpu.SemaphoreType.DMA((N_STEPS, 2)),
            pltpu.SemaphoreType.DMA((N_STEPS, 2)),
        ],
        compiler_params=pltpu.CompilerParams(
            collective_id=0, vmem_limit_bytes=56 * 1024 * 1024),
    )(Q, K, V)


# device time: 81273 ns/iter; 1.0840x vs baseline; 1.0840x over previous
import jax
import jax.numpy as jnp
from jax import lax
from jax.experimental import pallas as pl
from jax.experimental.pallas import tpu as pltpu

B, SQ, H, D = 4, 32, 8, 128
SKV_LOCAL = 4096
N_REPL = 4
SKV_Q = SKV_LOCAL // N_REPL
SCALE = D ** -0.5
N_STEPS = 3
NT = B * H


def kernel(Q, K, V):
    def body(q_ref, k_hbm, v_hbm, out_ref,
             kbuf, vbuf, o_acc, ml_acc, o_send, o_recv, ml_send, ml_recv,
             load_sems, send_sems, recv_sems):
        mx = lax.axis_index("x")
        my = lax.axis_index("y")
        mz = lax.axis_index("z")
        r = mx * 2 + mz
        start = r * SKV_Q

        def start_load(t):
            b, h = t // H, t % H
            slot = t % 2
            pltpu.make_async_copy(
                k_hbm.at[b, pl.ds(start, SKV_Q), h, :],
                kbuf.at[slot], load_sems.at[slot, 0]).start()
            pltpu.make_async_copy(
                v_hbm.at[b, pl.ds(start, SKV_Q), h, :],
                vbuf.at[slot], load_sems.at[slot, 1]).start()

        def wait_load(t):
            b, h = t // H, t % H
            slot = t % 2
            pltpu.make_async_copy(
                k_hbm.at[b, pl.ds(start, SKV_Q), h, :],
                kbuf.at[slot], load_sems.at[slot, 0]).wait()
            pltpu.make_async_copy(
                v_hbm.at[b, pl.ds(start, SKV_Q), h, :],
                vbuf.at[slot], load_sems.at[slot, 1]).wait()

        start_load(0)

        partners = [
            (mx, my, 1 - mz),
            (mx, 1 - my, mz),
            (1 - mx, my, mz),
        ]
        barrier = pltpu.get_barrier_semaphore()
        for p in partners:
            pl.semaphore_signal(barrier, inc=1, device_id=p,
                                device_id_type=pl.DeviceIdType.MESH)
        pl.semaphore_wait(barrier, N_STEPS)

        for t in range(NT):
            b, h = t // H, t % H
            slot = t % 2
            wait_load(t)
            if t + 1 < NT:
                start_load(t + 1)
            q = q_ref[b, :, h, :].astype(jnp.bfloat16)
            k = kbuf[slot].astype(jnp.bfloat16)
            v = vbuf[slot].astype(jnp.bfloat16)
            s = lax.dot_general(
                q, k, (((1,), (1,)), ((), ())),
                preferred_element_type=jnp.float32) * SCALE
            m = jnp.max(s, axis=1, keepdims=True)
            p = jnp.exp(s - m)
            l = jnp.sum(p, axis=1, keepdims=True)
            o = lax.dot_general(
                p.astype(jnp.bfloat16), v, (((1,), (0,)), ((), ())),
                preferred_element_type=jnp.float32)
            o_acc[b, h] = o
            ml_acc[b, h, :, 0:1] = m
            ml_acc[b, h, :, 1:2] = l

        for step in range(N_STEPS):
            o_send[step] = o_acc[...].astype(jnp.bfloat16)
            ml_send[step] = ml_acc[...]
            rdma_o = pltpu.make_async_remote_copy(
                src_ref=o_send.at[step], dst_ref=o_recv.at[step],
                send_sem=send_sems.at[step, 0], recv_sem=recv_sems.at[step, 0],
                device_id=partners[step],
                device_id_type=pl.DeviceIdType.MESH)
            rdma_ml = pltpu.make_async_remote_copy(
                src_ref=ml_send.at[step], dst_ref=ml_recv.at[step],
                send_sem=send_sems.at[step, 1], recv_sem=recv_sems.at[step, 1],
                device_id=partners[step],
                device_id_type=pl.DeviceIdType.MESH)
            rdma_o.start()
            rdma_ml.start()
            rdma_o.wait()
            rdma_ml.wait()

            m_a = ml_acc[:, :, :, 0:1]
            l_a = ml_acc[:, :, :, 1:2]
            m_b = ml_recv[step, :, :, :, 0:1]
            l_b = ml_recv[step, :, :, :, 1:2]
            m_n = jnp.maximum(m_a, m_b)
            ea = jnp.exp(m_a - m_n)
            eb = jnp.exp(m_b - m_n)
            o_acc[...] = ea * o_acc[...] + eb * o_recv[step].astype(jnp.float32)
            ml_acc[:, :, :, 0:1] = m_n
            ml_acc[:, :, :, 1:2] = ea * l_a + eb * l_b

        for b in range(B):
            for h in range(H):
                out_ref[b, :, h, :] = o_acc[b, h] / ml_acc[b, h, :, 1:2]

    return pl.pallas_call(
        body,
        out_shape=jax.ShapeDtypeStruct((B, SQ, H, D), jnp.float32),
        in_specs=[
            pl.BlockSpec(memory_space=pltpu.VMEM),
            pl.BlockSpec(memory_space=pl.ANY),
            pl.BlockSpec(memory_space=pl.ANY),
        ],
        out_specs=pl.BlockSpec(memory_space=pltpu.VMEM),
        scratch_shapes=[
            pltpu.VMEM((2, SKV_Q, D), jnp.float32),
            pltpu.VMEM((2, SKV_Q, D), jnp.float32),
            pltpu.VMEM((B, H, SQ, D), jnp.float32),
            pltpu.VMEM((B, H, SQ, 2), jnp.float32),
            pltpu.VMEM((N_STEPS, B, H, SQ, D), jnp.bfloat16),
            pltpu.VMEM((N_STEPS, B, H, SQ, D), jnp.bfloat16),
            pltpu.VMEM((N_STEPS, B, H, SQ, 2), jnp.float32),
            pltpu.VMEM((N_STEPS, B, H, SQ, 2), jnp.float32),
            pltpu.SemaphoreType.DMA((2, 2)),
            pltpu.SemaphoreType.DMA((N_STEPS, 2)),
            pltpu.SemaphoreType.DMA((N_STEPS, 2)),
        ],
        compiler_params=pltpu.CompilerParams(
            collective_id=0, vmem_limit_bytes=56 * 1024 * 1024),
    )(Q, K, V)
